# baseline (speedup 1.0000x reference)
import jax
import jax.numpy as jnp
from jax import lax
from jax.experimental import pallas as pl
from jax.experimental.pallas import tpu as pltpu

_R = 160
_L = 128
_N = 20000
_P = _R * _L
_TOPK = 400
_C = 512
_KEEP = 200
_CONF = 0.5
_NMS_T = 0.5
_NEG = -1e9
_BIG = 3.4e38


def _body(l0, l1, l2, l3, c0, c1, p0, p1, p2, p3, out_ref,
          sx1, sy1, sx2, sy2, scs, scl, scv, smk):
    lx = l0[0]
    ly = l1[0]
    lw = l2[0]
    lh = l3[0]
    s0 = c0[0]
    s1 = c1[0]
    px = p0[...]
    py = p1[...]
    pw = p2[...]
    ph = p3[...]

    cx = px + lx * 0.1 * pw
    cy = py + ly * 0.1 * ph
    w = pw * jnp.exp(lw * 0.2)
    h = ph * jnp.exp(lh * 0.2)
    x1 = cx - w / 2.0
    y1 = cy - h / 2.0
    x2 = w + x1
    y2 = h + y1

    scores = jnp.maximum(s0, s1)
    labels = jnp.where(s1 > s0, 1.0, 0.0)
    validf = jnp.where(scores > _CONF, 1.0, 0.0)
    masked0 = jnp.where(scores > _CONF, scores, _NEG)

    row_i = lax.broadcasted_iota(jnp.int32, (_R, _L), 0)
    col_i = lax.broadcasted_iota(jnp.int32, (_R, _L), 1)
    flat_i = row_i * _L + col_i
    lane_c = lax.broadcasted_iota(jnp.int32, (1, _C), 1)
    sub_c = lax.broadcasted_iota(jnp.int32, (_C, 1), 0)
    lane_o = lax.broadcasted_iota(jnp.int32, (1, _L), 1)

    zr = jnp.zeros((1, _C), jnp.float32)
    zc = jnp.zeros((_C, 1), jnp.float32)

    sx1[...] = x1
    sy1[...] = y1
    sx2[...] = x2
    sy2[...] = y2
    scs[...] = scores
    scl[...] = labels
    scv[...] = validf
    smk[...] = masked0

    flat8 = (lax.broadcasted_iota(jnp.int32, (8, _L), 0) * _L +
             lax.broadcasted_iota(jnp.int32, (8, _L), 1))
    gmax0 = jnp.full((1, _L), -_BIG, jnp.float32)
    for g in range(_R // 8):
        gmax0 = jnp.where(lane_o == g,
                          jnp.max(masked0[8 * g:8 * (g + 1), :]), gmax0)

    def topk_body(j, carry):
        (gmax, rx1, ry1, rx2, ry2, rcs, rcl, rcv,
         gx1, gy1, gx2, gy2, gcs, gcl, gcv) = carry
        m = jnp.max(gmax)
        g = jnp.min(jnp.where(gmax == m, lane_o, jnp.int32(2147483647)))
        r0 = g * 8
        blk = smk[pl.ds(r0, 8), :]
        fl = jnp.min(jnp.where(blk == m, flat8, jnp.int32(2147483647)))
        rl = fl // _L
        c = fl - rl * _L
        r = r0 + rl
        oh = jnp.where(lane_o == c, 1.0, 0.0)

        def gat(a):
            return jnp.sum(a[pl.ds(r, 1), :] * oh)

        vx1 = gat(sx1)
        vy1 = gat(sy1)
        vx2 = gat(sx2)
        vy2 = gat(sy2)
        vcs = gat(scs)
        vcl = gat(scl)
        vcv = gat(scv)
        smk[pl.ds(r, 1), :] = jnp.where(lane_o == c, -_BIG,
                                        smk[pl.ds(r, 1), :])
        gmax = jnp.where(lane_o == g, jnp.max(smk[pl.ds(r0, 8), :]), gmax)
        wr = lane_c == j
        wc = sub_c == j
        rx1 = jnp.where(wr, vx1, rx1)
        ry1 = jnp.where(wr, vy1, ry1)
        rx2 = jnp.where(wr, vx2, rx2)
        ry2 = jnp.where(wr, vy2, ry2)
        rcs = jnp.where(wr, vcs, rcs)
        rcl = jnp.where(wr, vcl, rcl)
        rcv = jnp.where(wr, vcv, rcv)
        gx1 = jnp.where(wc, vx1, gx1)
        gy1 = jnp.where(wc, vy1, gy1)
        gx2 = jnp.where(wc, vx2, gx2)
        gy2 = jnp.where(wc, vy2, gy2)
        gcs = jnp.where(wc, vcs, gcs)
        gcl = jnp.where(wc, vcl, gcl)
        gcv = jnp.where(wc, vcv, gcv)
        return (gmax, rx1, ry1, rx2, ry2, rcs, rcl, rcv,
                gx1, gy1, gx2, gy2, gcs, gcl, gcv)

    (_, rx1, ry1, rx2, ry2, rcs, rcl, rcv,
     gx1, gy1, gx2, gy2, gcs, gcl, gcv) = lax.fori_loop(
        0, _TOPK, topk_body,
        (gmax0, zr, zr, zr, zr, zr, zr, zr, zc, zc, zc, zc, zc, zc, zc))

    area_c = (gx2 - gx1) * (gy2 - gy1)
    area_r = (rx2 - rx1) * (ry2 - ry1)
    ltx = jnp.maximum(gx1, rx1)
    lty = jnp.maximum(gy1, ry1)
    rbx = jnp.minimum(gx2, rx2)
    rby = jnp.minimum(gy2, ry2)
    iw = jnp.maximum(rbx - ltx, 0.0)
    ih = jnp.maximum(rby - lty, 0.0)
    inter = iw * ih
    union = area_c + area_r - inter
    iou = inter / jnp.maximum(union, 1e-9)

    s_m = jnp.where((iou > _NMS_T) & (sub_c < lane_c), 1.0, 0.0)
    s_t = jnp.where((iou > _NMS_T) & (lane_c < sub_c), 1.0, 0.0)

    def nms_cond(carry):
        kr, kc, kprev, it = carry
        return jnp.logical_and(jnp.sum(jnp.abs(kr - kprev)) > 0.0,
                               it < _TOPK + 2)

    def nms_step(carry):
        kr, kc, _, it = carry
        sup_r = jnp.sum(s_m * kc, axis=0, keepdims=True)
        sup_c = jnp.sum(s_t * kr, axis=1, keepdims=True)
        krn = rcv * jnp.where(sup_r > 0.0, 0.0, 1.0)
        kcn = gcv * jnp.where(sup_c > 0.0, 0.0, 1.0)
        return krn, kcn, kr, it + 1

    keep, keepc, _, _ = lax.while_loop(
        nms_cond, nms_step, (rcv, gcv, rcv + 2.0, jnp.int32(0)))

    keyr = jnp.where(keep > 0.0, rcs, 1e9)
    keyc = jnp.where(keepc > 0.0, gcs, 1e9)
    less = jnp.where((keyc < keyr) | ((keyc == keyr) & (sub_c < lane_c)),
                     1.0, 0.0)
    rank = jnp.sum(less, axis=0, keepdims=True)
    riota = lax.broadcasted_iota(
        jnp.int32, (_KEEP, 1), 0).astype(jnp.float32)
    perm = jnp.where(rank == riota, 1.0, 0.0)
    fields = keepc * (
        gcl * jnp.where(lane_o == 0, 1.0, 0.0) +
        gcs * jnp.where(lane_o == 1, 1.0, 0.0) +
        gx1 * jnp.where(lane_o == 2, 1.0, 0.0) +
        gy1 * jnp.where(lane_o == 3, 1.0, 0.0) +
        gx2 * jnp.where(lane_o == 4, 1.0, 0.0) +
        gy2 * jnp.where(lane_o == 5, 1.0, 0.0))
    out_ref[0] = jnp.dot(perm, fields, precision=lax.Precision.HIGHEST,
                         preferred_element_type=jnp.float32)


@jax.jit
def _run(*args):
    bspec = pl.BlockSpec((1, _R, _L), lambda b: (b, 0, 0))
    pspec = pl.BlockSpec((_R, _L), lambda b: (0, 0))
    return pl.pallas_call(
        _body,
        grid=(8,),
        in_specs=[bspec] * 6 + [pspec] * 4,
        out_specs=pl.BlockSpec((1, _KEEP, _L), lambda b: (b, 0, 0)),
        out_shape=jax.ShapeDtypeStruct((8, _KEEP, _L), jnp.float32),
        scratch_shapes=[pltpu.VMEM((_R, _L), jnp.float32)] * 8,
        compiler_params=pltpu.CompilerParams(
            dimension_semantics=("parallel",)),
    )(*args)


@jax.jit
def kernel(predictions, priors):
    pad = _P - _N
    comps = [
        jnp.pad(predictions[:, :, i], ((0, 0), (0, pad))).reshape(8, _R, _L)
        for i in range(6)
    ]
    prs = [
        jnp.pad(priors[:, i], (0, pad)).reshape(_R, _L) for i in range(4)
    ]
    out = _run(*comps, *prs)
    return out[:, :, :6]


# speedup vs baseline: 1.1935x; 1.1935x over previous
import jax
import jax.numpy as jnp
from jax import lax
from jax.experimental import pallas as pl
from jax.experimental.pallas import tpu as pltpu

_R = 160
_L = 128
_N = 20000
_P = _R * _L
_TOPK = 400
_C = 512
_KEEP = 200
_CONF = 0.5
_NMS_T = 0.5
_NEG = -1e9
_BIG = 3.4e38


def _body(l0, l1, l2, l3, c0, c1, p0, p1, p2, p3, out_ref,
          sx1, sy1, sx2, sy2, scs, scl, scv, smk):
    lx = l0[0]
    ly = l1[0]
    lw = l2[0]
    lh = l3[0]
    s0 = c0[0]
    s1 = c1[0]
    px = p0[...]
    py = p1[...]
    pw = p2[...]
    ph = p3[...]

    cx = px + lx * 0.1 * pw
    cy = py + ly * 0.1 * ph
    w = pw * jnp.exp(lw * 0.2)
    h = ph * jnp.exp(lh * 0.2)
    x1 = cx - w / 2.0
    y1 = cy - h / 2.0
    x2 = w + x1
    y2 = h + y1

    scores = jnp.maximum(s0, s1)
    labels = jnp.where(s1 > s0, 1.0, 0.0)
    validf = jnp.where(scores > _CONF, 1.0, 0.0)
    masked0 = jnp.where(scores > _CONF, scores, _NEG)

    row_i = lax.broadcasted_iota(jnp.int32, (_R, _L), 0)
    col_i = lax.broadcasted_iota(jnp.int32, (_R, _L), 1)
    flat_i = row_i * _L + col_i
    lane_c = lax.broadcasted_iota(jnp.int32, (1, _C), 1)
    sub_c = lax.broadcasted_iota(jnp.int32, (_C, 1), 0)
    lane_o = lax.broadcasted_iota(jnp.int32, (1, _L), 1)

    zr = jnp.zeros((1, _C), jnp.float32)
    zc = jnp.zeros((_C, 1), jnp.float32)

    sx1[...] = x1
    sy1[...] = y1
    sx2[...] = x2
    sy2[...] = y2
    scs[...] = scores
    scl[...] = labels
    scv[...] = validf
    smk[...] = masked0

    def topk_body(j, carry):
        (rx1, ry1, rx2, ry2, rcs, rcl, rcv,
         gx1, gy1, gx2, gy2, gcs, gcl, gcv) = carry
        mk = smk[...]
        m = jnp.max(mk)
        fi = jnp.min(jnp.where(mk == m, flat_i, jnp.int32(2147483647)))
        r = fi // _L
        c = fi - r * _L
        oh = jnp.where(lane_o == c, 1.0, 0.0)

        def gat(a):
            return jnp.sum(a[pl.ds(r, 1), :] * oh)

        vx1 = gat(sx1)
        vy1 = gat(sy1)
        vx2 = gat(sx2)
        vy2 = gat(sy2)
        vcs = gat(scs)
        vcl = gat(scl)
        vcv = gat(scv)
        smk[pl.ds(r, 1), :] = jnp.where(lane_o == c, -_BIG,
                                        smk[pl.ds(r, 1), :])
        wr = lane_c == j
        wc = sub_c == j
        rx1 = jnp.where(wr, vx1, rx1)
        ry1 = jnp.where(wr, vy1, ry1)
        rx2 = jnp.where(wr, vx2, rx2)
        ry2 = jnp.where(wr, vy2, ry2)
        rcs = jnp.where(wr, vcs, rcs)
        rcl = jnp.where(wr, vcl, rcl)
        rcv = jnp.where(wr, vcv, rcv)
        gx1 = jnp.where(wc, vx1, gx1)
        gy1 = jnp.where(wc, vy1, gy1)
        gx2 = jnp.where(wc, vx2, gx2)
        gy2 = jnp.where(wc, vy2, gy2)
        gcs = jnp.where(wc, vcs, gcs)
        gcl = jnp.where(wc, vcl, gcl)
        gcv = jnp.where(wc, vcv, gcv)
        return (rx1, ry1, rx2, ry2, rcs, rcl, rcv,
                gx1, gy1, gx2, gy2, gcs, gcl, gcv)

    (rx1, ry1, rx2, ry2, rcs, rcl, rcv,
     gx1, gy1, gx2, gy2, gcs, gcl, gcv) = lax.fori_loop(
        0, _TOPK, topk_body,
        (zr, zr, zr, zr, zr, zr, zr, zc, zc, zc, zc, zc, zc, zc))

    area_c = (gx2 - gx1) * (gy2 - gy1)
    area_r = (rx2 - rx1) * (ry2 - ry1)
    ltx = jnp.maximum(gx1, rx1)
    lty = jnp.maximum(gy1, ry1)
    rbx = jnp.minimum(gx2, rx2)
    rby = jnp.minimum(gy2, ry2)
    iw = jnp.maximum(rbx - ltx, 0.0)
    ih = jnp.maximum(rby - lty, 0.0)
    inter = iw * ih
    union = area_c + area_r - inter
    iou = inter / jnp.maximum(union, 1e-9)

    s_m = jnp.where((iou > _NMS_T) & (sub_c < lane_c), 1.0, 0.0)
    s_t = jnp.where((iou > _NMS_T) & (lane_c < sub_c), 1.0, 0.0)

    def nms_cond(carry):
        kr, kc, kprev, it = carry
        return jnp.logical_and(jnp.sum(jnp.abs(kr - kprev)) > 0.0,
                               it < _TOPK + 2)

    def nms_step(carry):
        kr, kc, _, it = carry
        sup_r = jnp.sum(s_m * kc, axis=0, keepdims=True)
        sup_c = jnp.sum(s_t * kr, axis=1, keepdims=True)
        krn = rcv * jnp.where(sup_r > 0.0, 0.0, 1.0)
        kcn = gcv * jnp.where(sup_c > 0.0, 0.0, 1.0)
        return krn, kcn, kr, it + 1

    keep, keepc, _, _ = lax.while_loop(
        nms_cond, nms_step, (rcv, gcv, rcv + 2.0, jnp.int32(0)))

    keyr = jnp.where(keep > 0.0, rcs, 1e9)
    keyc = jnp.where(keepc > 0.0, gcs, 1e9)
    less = jnp.where((keyc < keyr) | ((keyc == keyr) & (sub_c < lane_c)),
                     1.0, 0.0)
    rank = jnp.sum(less, axis=0, keepdims=True)
    riota = lax.broadcasted_iota(
        jnp.int32, (_KEEP, 1), 0).astype(jnp.float32)
    perm = jnp.where(rank == riota, 1.0, 0.0)
    fields = keepc * (
        gcl * jnp.where(lane_o == 0, 1.0, 0.0) +
        gcs * jnp.where(lane_o == 1, 1.0, 0.0) +
        gx1 * jnp.where(lane_o == 2, 1.0, 0.0) +
        gy1 * jnp.where(lane_o == 3, 1.0, 0.0) +
        gx2 * jnp.where(lane_o == 4, 1.0, 0.0) +
        gy2 * jnp.where(lane_o == 5, 1.0, 0.0))
    out_ref[0] = jnp.dot(perm, fields, precision=lax.Precision.HIGHEST,
                         preferred_element_type=jnp.float32)


@jax.jit
def _run(*args):
    bspec = pl.BlockSpec((1, _R, _L), lambda b: (b, 0, 0))
    pspec = pl.BlockSpec((_R, _L), lambda b: (0, 0))
    return pl.pallas_call(
        _body,
        grid=(8,),
        in_specs=[bspec] * 6 + [pspec] * 4,
        out_specs=pl.BlockSpec((1, _KEEP, _L), lambda b: (b, 0, 0)),
        out_shape=jax.ShapeDtypeStruct((8, _KEEP, _L), jnp.float32),
        scratch_shapes=[pltpu.VMEM((_R, _L), jnp.float32)] * 8,
        compiler_params=pltpu.CompilerParams(
            dimension_semantics=("parallel",)),
    )(*args)


@jax.jit
def kernel(predictions, priors):
    pad = _P - _N
    comps = [
        jnp.pad(predictions[:, :, i], ((0, 0), (0, pad))).reshape(8, _R, _L)
        for i in range(6)
    ]
    prs = [
        jnp.pad(priors[:, i], (0, pad)).reshape(_R, _L) for i in range(4)
    ]
    out = _run(*comps, *prs)
    return out[:, :, :6]
